# baseline (device time: 399322 ns/iter reference)
import jax
import jax.numpy as jnp
from jax import lax
from jax.experimental import pallas as pl
from jax.experimental.pallas import tpu as pltpu

N_DEV = 32
R_HOPS = N_DEV // 2
L_HOPS = N_DEV - 1 - R_HOPS

_PLANE = [(0, 0), (1, 0), (1, 1), (0, 1), (0, 2), (1, 2), (1, 3), (0, 3)]
_COORD_OF_LOGICAL = [
    (x, y, z) for z in range(4) for (x, y) in _PLANE
]
_LOGICAL_OF_COORD = {c: p for p, c in enumerate(_COORD_OF_LOGICAL)}

_PATH_YZ = [
    (0, 0), (1, 0), (2, 0), (3, 0),
    (3, 1), (2, 1), (1, 1), (0, 1),
    (0, 2), (1, 2), (2, 2), (3, 2),
    (3, 3), (2, 3), (1, 3), (0, 3),
]
_CYCLE_COORDS = [(0, y, z) for (y, z) in _PATH_YZ] + [
    (1, y, z) for (y, z) in reversed(_PATH_YZ)
]
_RING = [_LOGICAL_OF_COORD[c] for c in _CYCLE_COORDS]
_RING_POS = [0] * N_DEV
for _rp, _p in enumerate(_RING):
    _RING_POS[_p] = _rp


def _silu(y):
    return y * jax.nn.sigmoid(y)


def kernel(x, w_mat):
    m_per, k = x.shape
    _, n_per = w_mat.shape

    my = lax.axis_index("i")
    ring = jnp.asarray(_RING, dtype=jnp.int32)
    rp = jnp.asarray(_RING_POS, dtype=jnp.int32)[my]
    nbrs = jnp.stack([
        ring[(rp + 1) % N_DEV],
        ring[(rp - 1) % N_DEV],
    ]).astype(jnp.int32)
    origins_f = ring[(rp - jnp.arange(R_HOPS + 1)) % N_DEV].astype(jnp.int32)
    origins_b = ring[(rp + jnp.arange(L_HOPS + 1)) % N_DEV].astype(jnp.int32)

    def body(
        nbrs_ref, orf_ref, orb_ref, x_ref, w_ref, out_ref,
        f_comm, b_comm,
        f_send_sems, f_recv_sems, b_send_sems, b_recv_sems,
        f_credit, b_credit,
    ):
        succ = nbrs_ref[0]
        pred = nbrs_ref[1]

        barrier_sem = pltpu.get_barrier_semaphore()
        for nbr in (succ, pred):
            pl.semaphore_signal(
                barrier_sem, inc=1,
                device_id=(nbr,), device_id_type=pl.DeviceIdType.MESH,
            )
        pl.semaphore_wait(barrier_sem, 2)

        def compute_half(arr, origin, hi):
            blk = jnp.dot(arr, w_ref[...], preferred_element_type=jnp.float32)
            rows = m_per // 2
            out_ref[pl.ds(origin * m_per + hi * rows, rows), :] = _silu(blk)

        P = 4
        half = m_per // P

        def piece(ref2d):
            return lambda j: ref2d.at[pl.ds(j * half, half), :]

        def comm_piece(comm, slot):
            return lambda j: comm.at[slot, pl.ds(j * half, half), :]

        def send_desc(src_p, comm, slot_s, slot_r, send_sems, recv_sems, dst, j):
            return pltpu.make_async_remote_copy(
                src_ref=src_p(j),
                dst_ref=comm_piece(comm, slot_r)(j),
                send_sem=send_sems.at[slot_s, j],
                recv_sem=recv_sems.at[slot_r, j],
                device_id=(dst,),
                device_id_type=pl.DeviceIdType.MESH,
            )

        def recv_desc(comm, slot, send_sems, recv_sems, dst, j):
            p = comm_piece(comm, slot)(j)
            return pltpu.make_async_remote_copy(
                src_ref=p, dst_ref=p,
                send_sem=send_sems.at[slot, j],
                recv_sem=recv_sems.at[slot, j],
                device_id=(dst,),
                device_id_type=pl.DeviceIdType.MESH,
            )

        for h in range(R_HOPS):
            s, r = h % 2, (h + 1) % 2
            if h >= 2:
                pl.semaphore_wait(f_credit, 1)
                if h <= L_HOPS - 1:
                    pl.semaphore_wait(b_credit, 1)
            f_src = piece(x_ref) if h == 0 else comm_piece(f_comm, s)
            b_src = piece(x_ref) if h == 0 else comm_piece(b_comm, s)
            rows = m_per // 2
            f_sends, b_sends = [], []
            for j in range(P):
                if h > 0:
                    recv_desc(f_comm, s, f_send_sems, f_recv_sems, succ, j).wait_recv()
                d = send_desc(f_src, f_comm, s, r, f_send_sems, f_recv_sems, succ, j)
                d.start()
                f_sends.append(d)
                if 1 <= h <= L_HOPS:
                    recv_desc(b_comm, s, b_send_sems, b_recv_sems, pred, j).wait_recv()
                if h <= L_HOPS - 1:
                    d = send_desc(b_src, b_comm, s, r, b_send_sems, b_recv_sems, pred, j)
                    d.start()
                    b_sends.append(d)
                if j % 2 == 1:
                    hi = j // 2
                    rs = slice(hi * rows, (hi + 1) * rows)
                    if h == 0:
                        compute_half(x_ref[rs, :], orf_ref[0], hi)
                    else:
                        compute_half(f_comm[s, rs, :], orf_ref[h], hi)
                        compute_half(b_comm[s, rs, :], orb_ref[h], hi)
            for d in f_sends:
                d.wait_send()
            if 1 <= h <= R_HOPS - 2:
                pl.semaphore_signal(
                    f_credit, inc=1,
                    device_id=(pred,), device_id_type=pl.DeviceIdType.MESH,
                )
            for d in b_sends:
                d.wait_send()
            if 1 <= h <= L_HOPS - 2:
                pl.semaphore_signal(
                    b_credit, inc=1,
                    device_id=(succ,), device_id_type=pl.DeviceIdType.MESH,
                )

        s_last = R_HOPS % 2
        rows = m_per // 2
        for j in range(P):
            recv_desc(f_comm, s_last, f_send_sems, f_recv_sems, succ, j).wait_recv()
            if j % 2 == 1:
                hi = j // 2
                rs = slice(hi * rows, (hi + 1) * rows)
                compute_half(f_comm[s_last, rs, :], orf_ref[R_HOPS], hi)

    out_shape = jax.ShapeDtypeStruct((N_DEV * m_per, n_per), jnp.float32)
    return pl.pallas_call(
        body,
        out_shape=out_shape,
        in_specs=[
            pl.BlockSpec(memory_space=pltpu.SMEM),
            pl.BlockSpec(memory_space=pltpu.SMEM),
            pl.BlockSpec(memory_space=pltpu.SMEM),
            pl.BlockSpec(memory_space=pltpu.VMEM),
            pl.BlockSpec(memory_space=pltpu.VMEM),
        ],
        out_specs=pl.BlockSpec(memory_space=pltpu.VMEM),
        scratch_shapes=[
            pltpu.VMEM((2, m_per, k), jnp.float32),
            pltpu.VMEM((2, m_per, k), jnp.float32),
            pltpu.SemaphoreType.DMA((2, 4)),
            pltpu.SemaphoreType.DMA((2, 4)),
            pltpu.SemaphoreType.DMA((2, 4)),
            pltpu.SemaphoreType.DMA((2, 4)),
            pltpu.SemaphoreType.REGULAR,
            pltpu.SemaphoreType.REGULAR,
        ],
        compiler_params=pltpu.CompilerParams(collective_id=0),
    )(nbrs, origins_f, origins_b, x, w_mat)


# device time: 389141 ns/iter; 1.0262x vs baseline; 1.0262x over previous
import jax
import jax.numpy as jnp
from jax import lax
from jax.experimental import pallas as pl
from jax.experimental.pallas import tpu as pltpu

N_DEV = 32
R_HOPS = N_DEV // 2
L_HOPS = N_DEV - 1 - R_HOPS

_PLANE = [(0, 0), (1, 0), (1, 1), (0, 1), (0, 2), (1, 2), (1, 3), (0, 3)]
_COORD_OF_LOGICAL = [
    (x, y, z) for z in range(4) for (x, y) in _PLANE
]
_LOGICAL_OF_COORD = {c: p for p, c in enumerate(_COORD_OF_LOGICAL)}

_PATH_YZ = [
    (0, 0), (1, 0), (2, 0), (3, 0),
    (3, 1), (2, 1), (1, 1), (0, 1),
    (0, 2), (1, 2), (2, 2), (3, 2),
    (3, 3), (2, 3), (1, 3), (0, 3),
]
_CYCLE_COORDS = [(0, y, z) for (y, z) in _PATH_YZ] + [
    (1, y, z) for (y, z) in reversed(_PATH_YZ)
]
_RING = [_LOGICAL_OF_COORD[c] for c in _CYCLE_COORDS]
_RING_POS = [0] * N_DEV
for _rp, _p in enumerate(_RING):
    _RING_POS[_p] = _rp


def _silu(y):
    return y * jax.nn.sigmoid(y)


def kernel(x, w_mat):
    m_per, k = x.shape
    _, n_per = w_mat.shape

    my = lax.axis_index("i")
    ring = jnp.asarray(_RING, dtype=jnp.int32)
    rp = jnp.asarray(_RING_POS, dtype=jnp.int32)[my]
    nbrs = jnp.stack([
        ring[(rp + 1) % N_DEV],
        ring[(rp - 1) % N_DEV],
    ]).astype(jnp.int32)
    origins_f = ring[(rp - jnp.arange(R_HOPS + 1)) % N_DEV].astype(jnp.int32)
    origins_b = ring[(rp + jnp.arange(L_HOPS + 1)) % N_DEV].astype(jnp.int32)

    def body(
        nbrs_ref, orf_ref, orb_ref, x_ref, w_ref, out_ref,
        f_comm, b_comm,
        f_send_sems, f_recv_sems, b_send_sems, b_recv_sems,
        f_credit, b_credit,
    ):
        succ = nbrs_ref[0]
        pred = nbrs_ref[1]

        barrier_sem = pltpu.get_barrier_semaphore()
        for nbr in (succ, pred):
            pl.semaphore_signal(
                barrier_sem, inc=1,
                device_id=(nbr,), device_id_type=pl.DeviceIdType.MESH,
            )
        pl.semaphore_wait(barrier_sem, 2)

        def compute_half(arr, origin, hi):
            blk = jnp.dot(arr, w_ref[...], preferred_element_type=jnp.float32)
            rows = m_per // 2
            out_ref[pl.ds(origin * m_per + hi * rows, rows), :] = _silu(blk)

        P = 4
        half = m_per // P

        def piece(ref2d):
            return lambda j: ref2d.at[pl.ds(j * half, half), :]

        def comm_piece(comm, slot):
            return lambda j: comm.at[slot, pl.ds(j * half, half), :]

        def send_desc(src_p, comm, slot_s, slot_r, send_sems, recv_sems, dst, j):
            return pltpu.make_async_remote_copy(
                src_ref=src_p(j),
                dst_ref=comm_piece(comm, slot_r)(j),
                send_sem=send_sems.at[slot_s, j],
                recv_sem=recv_sems.at[slot_r, j],
                device_id=(dst,),
                device_id_type=pl.DeviceIdType.MESH,
            )

        def recv_desc(comm, slot, send_sems, recv_sems, dst, j):
            p = comm_piece(comm, slot)(j)
            return pltpu.make_async_remote_copy(
                src_ref=p, dst_ref=p,
                send_sem=send_sems.at[slot, j],
                recv_sem=recv_sems.at[slot, j],
                device_id=(dst,),
                device_id_type=pl.DeviceIdType.MESH,
            )

        for h in range(R_HOPS):
            s, r = h % 2, (h + 1) % 2
            if h >= 2:
                pl.semaphore_wait(f_credit, 1)
                if h <= L_HOPS:
                    pl.semaphore_wait(b_credit, 1)
            f_src = piece(x_ref) if h == 0 else comm_piece(f_comm, s)
            b_src = piece(x_ref) if h == 0 else comm_piece(b_comm, s)
            rows = m_per // 2
            f_sends, b_sends = [], []
            for j in range(P):
                if h > 0:
                    recv_desc(f_comm, s, f_send_sems, f_recv_sems, succ, j).wait_recv()
                if h < R_HOPS - 1 or j < P // 2:
                    d = send_desc(f_src, f_comm, s, r, f_send_sems, f_recv_sems, succ, j)
                    d.start()
                    f_sends.append(d)
                if 1 <= h <= L_HOPS:
                    recv_desc(b_comm, s, b_send_sems, b_recv_sems, pred, j).wait_recv()
                if h <= L_HOPS - 1 or (h == L_HOPS and j >= P // 2):
                    d = send_desc(b_src, b_comm, s, r, b_send_sems, b_recv_sems, pred, j)
                    d.start()
                    b_sends.append(d)
                if j % 2 == 1:
                    hi = j // 2
                    rs = slice(hi * rows, (hi + 1) * rows)
                    if h == 0:
                        compute_half(x_ref[rs, :], orf_ref[0], hi)
                    else:
                        compute_half(f_comm[s, rs, :], orf_ref[h], hi)
                        compute_half(b_comm[s, rs, :], orb_ref[h], hi)
            for d in f_sends:
                d.wait_send()
            if 1 <= h <= R_HOPS - 2:
                pl.semaphore_signal(
                    f_credit, inc=1,
                    device_id=(pred,), device_id_type=pl.DeviceIdType.MESH,
                )
            for d in b_sends:
                d.wait_send()
            if 1 <= h <= L_HOPS - 1:
                pl.semaphore_signal(
                    b_credit, inc=1,
                    device_id=(succ,), device_id_type=pl.DeviceIdType.MESH,
                )

        s_last = R_HOPS % 2
        rows = m_per // 2
        for j in range(P // 2):
            recv_desc(f_comm, s_last, f_send_sems, f_recv_sems, succ, j).wait_recv()
        compute_half(f_comm[s_last, 0:rows, :], orf_ref[R_HOPS], 0)
        for j in range(P // 2, P):
            recv_desc(b_comm, s_last, b_send_sems, b_recv_sems, pred, j).wait_recv()
        compute_half(b_comm[s_last, rows:m_per, :], orf_ref[R_HOPS], 1)

    out_shape = jax.ShapeDtypeStruct((N_DEV * m_per, n_per), jnp.float32)
    return pl.pallas_call(
        body,
        out_shape=out_shape,
        in_specs=[
            pl.BlockSpec(memory_space=pltpu.SMEM),
            pl.BlockSpec(memory_space=pltpu.SMEM),
            pl.BlockSpec(memory_space=pltpu.SMEM),
            pl.BlockSpec(memory_space=pltpu.VMEM),
            pl.BlockSpec(memory_space=pltpu.VMEM),
        ],
        out_specs=pl.BlockSpec(memory_space=pltpu.VMEM),
        scratch_shapes=[
            pltpu.VMEM((2, m_per, k), jnp.float32),
            pltpu.VMEM((2, m_per, k), jnp.float32),
            pltpu.SemaphoreType.DMA((2, 4)),
            pltpu.SemaphoreType.DMA((2, 4)),
            pltpu.SemaphoreType.DMA((2, 4)),
            pltpu.SemaphoreType.DMA((2, 4)),
            pltpu.SemaphoreType.REGULAR,
            pltpu.SemaphoreType.REGULAR,
        ],
        compiler_params=pltpu.CompilerParams(collective_id=0),
    )(nbrs, origins_f, origins_b, x, w_mat)
